# baseline (speedup 1.0000x reference)
import functools

import jax
import jax.numpy as jnp
from jax import lax
from jax.experimental import pallas as pl
from jax.experimental.pallas import tpu as pltpu
from jax.experimental.pallas import tpu_sc as plsc

NC, NS, L = 2, 16, 16
NW = NC * NS
BATCH = 16384
D = 64
BPW = BATCH // NW
NG = BPW // L
DCH = D // L

_GATHER_DNUMS = lax.GatherDimensionNumbers(
    offset_dims=(), collapsed_slice_dims=(0,), start_index_map=(0,))


def _shuffle(v, perm):
    return lax.gather(v, perm.reshape(L, 1), _GATHER_DNUMS, (1,),
                      mode=lax.GatherScatterMode.PROMISE_IN_BOUNDS)


def _transe_body(head_hbm, rel_hbm, tail_hbm, ent3_hbm, rel3_hbm, out_hbm,
                 idx_h, idx_r, idx_t, bufs0, bufs1, bufs2, bufs3, out_v,
                 sem0, sem1, sem2, sem3):
    wid = lax.axis_index("s") * NC + lax.axis_index("c")
    base = wid * BPW
    bufs = (bufs0, bufs1, bufs2, bufs3)
    sems = (sem0, sem1, sem2, sem3)

    pltpu.sync_copy(head_hbm.at[pl.ds(base, BPW)], idx_h)
    pltpu.sync_copy(rel_hbm.at[pl.ds(base, BPW)], idx_r)
    pltpu.sync_copy(tail_hbm.at[pl.ds(base, BPW)], idx_t)

    def issue(g, buf, sem):
        vh = idx_h[pl.ds(g * L, L)]
        vr = idx_r[pl.ds(g * L, L)]
        vt = idx_t[pl.ds(g * L, L)]
        for j in range(L):
            dst = pl.ds(j, 1)
            eh, er, et = vh[j], vr[j], vt[j]
            pltpu.async_copy(
                ent3_hbm.at[eh >> 3, pl.ds(eh & 7, 1), :],
                buf.at[0, dst, :], sem)
            pltpu.async_copy(
                rel3_hbm.at[er >> 3, pl.ds(er & 7, 1), :],
                buf.at[1, dst, :], sem)
            pltpu.async_copy(
                ent3_hbm.at[et >> 3, pl.ds(et & 7, 1), :],
                buf.at[2, dst, :], sem)

    def drain(g, buf, sem):
        for q in range(6):
            pltpu.make_async_copy(
                ent3_hbm.at[0], buf.at[q % 3, pl.ds((q // 3) * 8, 8), :],
                sem).wait()

    lanes = lax.iota(jnp.int32, L)
    perms = {s: lanes ^ s for s in (8, 4, 2, 1)}
    masks = {s: (lanes & s) == 0 for s in (8, 4, 2, 1)}
    bitrev = (((lanes & 1) << 3) | ((lanes & 2) << 1)
              | ((lanes & 4) >> 1) | ((lanes & 8) >> 3))

    def combine(x, y, s):
        sel = jnp.where(masks[s], x, y)
        swp = jnp.where(masks[s], _shuffle(x, perms[s]), _shuffle(y, perms[s]))
        return sel + swp

    def compute(g, buf):
        parts = []
        for j in range(L):
            acc = jnp.abs(buf[0, j, pl.ds(0, L)] + buf[1, j, pl.ds(0, L)]
                          - buf[2, j, pl.ds(0, L)])
            for c in range(1, DCH):
                h = buf[0, j, pl.ds(c * L, L)]
                r = buf[1, j, pl.ds(c * L, L)]
                t = buf[2, j, pl.ds(c * L, L)]
                acc = acc + jnp.abs(h + r - t)
            parts.append(acc)
        for s in (8, 4, 2, 1):
            parts = [combine(parts[2 * i], parts[2 * i + 1], s)
                     for i in range(len(parts) // 2)]
        out_v[pl.ds(g * L, L)] = _shuffle(parts[0], bitrev)

    for p in range(4):
        issue(p, bufs[p], sems[p])

    def step(i, carry):
        for p in range(4):
            g = 4 * i + p
            drain(g, bufs[p], sems[p])
            compute(g, bufs[p])

            @pl.when(i < NG // 4 - 1)
            def _():
                issue(g + 4, bufs[p], sems[p])
        return carry

    lax.fori_loop(0, NG // 4, step, 0)

    pltpu.sync_copy(out_v, out_hbm.at[pl.ds(base, BPW)])


@functools.partial(
    pl.kernel,
    out_type=jax.ShapeDtypeStruct((BATCH,), jnp.float32),
    mesh=plsc.VectorSubcoreMesh(core_axis_name="c", subcore_axis_name="s"),
    scratch_types=[
        pltpu.VMEM((BPW,), jnp.int32),
        pltpu.VMEM((BPW,), jnp.int32),
        pltpu.VMEM((BPW,), jnp.int32),
        pltpu.VMEM((3, L, D), jnp.float32),
        pltpu.VMEM((3, L, D), jnp.float32),
        pltpu.VMEM((3, L, D), jnp.float32),
        pltpu.VMEM((3, L, D), jnp.float32),
        pltpu.VMEM((BPW,), jnp.float32),
        pltpu.SemaphoreType.DMA,
        pltpu.SemaphoreType.DMA,
        pltpu.SemaphoreType.DMA,
        pltpu.SemaphoreType.DMA,
    ],
)
def _transe_sc(head, relation, tail, ent3, rel3, out, *rest):
    _transe_body(head, relation, tail, ent3, rel3, out, *rest)


def kernel(head, relation, tail, entity_embeddings, relation_embeddings):
    ent3 = entity_embeddings.reshape(125000, 8, D)
    rel3 = relation_embeddings.reshape(125, 8, D)
    return _transe_sc(head, relation, tail, ent3, rel3)


# speedup vs baseline: 1.0342x; 1.0342x over previous
import functools

import jax
import jax.numpy as jnp
from jax import lax
from jax.experimental import pallas as pl
from jax.experimental.pallas import tpu as pltpu
from jax.experimental.pallas import tpu_sc as plsc

NC, NS, L = 2, 16, 16
NW = NC * NS
BATCH = 16384
D = 64
BPW = BATCH // NW
NG = BPW // L
DCH = D // L

_GATHER_DNUMS = lax.GatherDimensionNumbers(
    offset_dims=(), collapsed_slice_dims=(0,), start_index_map=(0,))


def _shuffle(v, perm):
    return lax.gather(v, perm.reshape(L, 1), _GATHER_DNUMS, (1,),
                      mode=lax.GatherScatterMode.PROMISE_IN_BOUNDS)


def _transe_body(head_hbm, rel_hbm, tail_hbm, ent3_hbm, rel3_hbm, out_hbm,
                 idx_h, idx_r, idx_t, bufs0, bufs1, out_v, sem0, sem1):
    wid = lax.axis_index("s") * NC + lax.axis_index("c")
    base = wid * BPW
    bufs = (bufs0, bufs1)
    sems = (sem0, sem1)

    pltpu.sync_copy(head_hbm.at[pl.ds(base, BPW)], idx_h)
    pltpu.sync_copy(rel_hbm.at[pl.ds(base, BPW)], idx_r)
    pltpu.sync_copy(tail_hbm.at[pl.ds(base, BPW)], idx_t)

    def issue(g, buf, sem):
        vh = idx_h[pl.ds(g * L, L)]
        vr = idx_r[pl.ds(g * L, L)]
        vt = idx_t[pl.ds(g * L, L)]
        for j in range(L):
            dst = pl.ds(j, 1)
            eh, er, et = vh[j], vr[j], vt[j]
            pltpu.async_copy(
                ent3_hbm.at[eh >> 3, pl.ds(eh & 7, 1), :],
                buf.at[0, dst, :], sem)
            pltpu.async_copy(
                rel3_hbm.at[er >> 3, pl.ds(er & 7, 1), :],
                buf.at[1, dst, :], sem)
            pltpu.async_copy(
                ent3_hbm.at[et >> 3, pl.ds(et & 7, 1), :],
                buf.at[2, dst, :], sem)

    def drain(g, buf, sem):
        for q in range(6):
            pltpu.make_async_copy(
                ent3_hbm.at[0], buf.at[q % 3, pl.ds((q // 3) * 8, 8), :],
                sem).wait()

    lanes = lax.iota(jnp.int32, L)
    perms = {s: lanes ^ s for s in (8, 4, 2, 1)}
    masks = {s: (lanes & s) == 0 for s in (8, 4, 2, 1)}
    bitrev = (((lanes & 1) << 3) | ((lanes & 2) << 1)
              | ((lanes & 4) >> 1) | ((lanes & 8) >> 3))

    def combine(x, y, s):
        sel = jnp.where(masks[s], x, y)
        swp = jnp.where(masks[s], _shuffle(x, perms[s]), _shuffle(y, perms[s]))
        return sel + swp

    def compute(g, buf):
        parts = []
        for j in range(L):
            acc = jnp.abs(buf[0, j, pl.ds(0, L)] + buf[1, j, pl.ds(0, L)]
                          - buf[2, j, pl.ds(0, L)])
            for c in range(1, DCH):
                h = buf[0, j, pl.ds(c * L, L)]
                r = buf[1, j, pl.ds(c * L, L)]
                t = buf[2, j, pl.ds(c * L, L)]
                acc = acc + jnp.abs(h + r - t)
            parts.append(acc)
        for s in (8, 4, 2, 1):
            parts = [combine(parts[2 * i], parts[2 * i + 1], s)
                     for i in range(len(parts) // 2)]
        out_v[pl.ds(g * L, L)] = _shuffle(parts[0], bitrev)

    issue(0, bufs[0], sems[0])
    issue(1, bufs[1], sems[1])

    def step(i, carry):
        for p in range(2):
            g = 2 * i + p
            drain(g, bufs[p], sems[p])
            compute(g, bufs[p])

            @pl.when(i < NG // 2 - 1)
            def _():
                issue(g + 2, bufs[p], sems[p])
        return carry

    lax.fori_loop(0, NG // 2, step, 0)

    pltpu.sync_copy(out_v, out_hbm.at[pl.ds(base, BPW)])


@functools.partial(
    pl.kernel,
    out_type=jax.ShapeDtypeStruct((BATCH,), jnp.float32),
    mesh=plsc.VectorSubcoreMesh(core_axis_name="c", subcore_axis_name="s"),
    scratch_types=[
        pltpu.VMEM((BPW,), jnp.int32),
        pltpu.VMEM((BPW,), jnp.int32),
        pltpu.VMEM((BPW,), jnp.int32),
        pltpu.VMEM((3, L, D), jnp.float32),
        pltpu.VMEM((3, L, D), jnp.float32),
        pltpu.VMEM((BPW,), jnp.float32),
        pltpu.SemaphoreType.DMA,
        pltpu.SemaphoreType.DMA,
    ],
)
def _transe_sc(head, relation, tail, ent3, rel3, out, *rest):
    _transe_body(head, relation, tail, ent3, rel3, out, *rest)


def kernel(head, relation, tail, entity_embeddings, relation_embeddings):
    ent3 = entity_embeddings.reshape(125000, 8, D)
    rel3 = relation_embeddings.reshape(125, 8, D)
    return _transe_sc(head, relation, tail, ent3, rel3)
